# baseline (speedup 1.0000x reference)
import functools

import jax
import jax.numpy as jnp
from jax import lax
from jax.experimental import pallas as pl
from jax.experimental.pallas import tpu as pltpu
from jax.experimental.pallas import tpu_sc as plsc

EMBED = 64
BATCH = 16384

_info = plsc.get_sparse_core_info()
_NC = _info.num_cores
_NS = _info.num_subcores
_NW = _NC * _NS
_BPW = BATCH // _NW

_mesh = plsc.VectorSubcoreMesh(core_axis_name="c", subcore_axis_name="s")


@functools.partial(
    pl.kernel,
    mesh=_mesh,
    out_type=jax.ShapeDtypeStruct((BATCH, 2 * EMBED), jnp.float32),
    scratch_types=[
        pltpu.VMEM((_BPW,), jnp.int32),
        pltpu.VMEM((_BPW,), jnp.int32),
        pltpu.VMEM((_BPW, 2 * EMBED), jnp.float32),
    ] + [pltpu.SemaphoreType.DMA] * 16,
)
def _embed_gather(ids_hbm, wu_hbm, wm_hbm, out_hbm,
                  idx_u, idx_m, combined, *sems):
    wid = lax.axis_index("s") * _NC + lax.axis_index("c")
    base = wid * _BPW

    pltpu.sync_copy(ids_hbm.at[0, pl.ds(base, _BPW)], idx_u)
    pltpu.sync_copy(ids_hbm.at[1, pl.ds(base, _BPW)], idx_m)

    @plsc.parallel_loop(0, _BPW // 16, 1, unroll=2)
    def _issue(g):
        vu = idx_u[pl.ds(g * 16, 16)]
        vm = idx_m[pl.ds(g * 16, 16)]
        for lane in range(16):
            j = g * 16 + lane
            pltpu.async_copy(wu_hbm.at[vu[lane]],
                             combined.at[j, pl.ds(0, EMBED)], sems[lane % 8])
            pltpu.async_copy(wm_hbm.at[vm[lane]],
                             combined.at[j, pl.ds(EMBED, EMBED)],
                             sems[8 + lane % 8])

    for k in range(16):
        pltpu.make_async_copy(out_hbm.at[pl.ds(0, 32), :],
                              combined.at[pl.ds(0, 32), :], sems[k]).wait()

    pltpu.sync_copy(combined, out_hbm.at[pl.ds(base, _BPW), :])


def kernel(input, W_user, W_movie):
    return _embed_gather(input, W_user, W_movie)
